# baseline (speedup 1.0000x reference)
import functools

import jax
import jax.numpy as jnp
from jax import lax
from jax.experimental import pallas as pl
from jax.experimental.pallas import tpu as pltpu
from jax.experimental.pallas import tpu_sc as plsc

BATCH = 16384
FIELDS = 26
EMBED = 64
TOTAL = BATCH * FIELDS
NUM_WORKERS = 32
PER_WORKER = TOTAL // NUM_WORKERS
CHUNK = 128
NCHUNKS = PER_WORKER // CHUNK


def _body(idx_hbm, table_hbm, out_hbm, idx_v, rows_v, gsem):
    wid = lax.axis_index("s") * 2 + lax.axis_index("c")
    base = wid * PER_WORKER
    pltpu.sync_copy(idx_hbm.at[wid], idx_v)

    def step(j, carry):
        row0 = pl.multiple_of(base + j * CHUNK, CHUNK)
        pltpu.async_copy(table_hbm.at[idx_v.at[j]], rows_v, gsem).wait()
        pltpu.sync_copy(rows_v, out_hbm.at[pl.ds(row0, CHUNK)])
        return carry

    lax.fori_loop(0, NCHUNKS, step, 0)


def kernel(indices, E):
    idx = indices.reshape(NUM_WORKERS, NCHUNKS, CHUNK).astype(jnp.int32)
    mesh = plsc.VectorSubcoreMesh(core_axis_name="c", subcore_axis_name="s")
    run = pl.kernel(
        _body,
        out_type=jax.ShapeDtypeStruct((TOTAL, EMBED), jnp.float32),
        mesh=mesh,
        scratch_types=[
            pltpu.VMEM((NCHUNKS, CHUNK), jnp.int32),
            pltpu.VMEM((CHUNK, EMBED), jnp.float32),
            pltpu.SemaphoreType.DMA,
        ],
        compiler_params=pltpu.CompilerParams(use_tc_tiling_on_sc=False),
    )
    out = run(idx, E)
    return out.reshape(BATCH, FIELDS, EMBED)


# speedup vs baseline: 1.0769x; 1.0769x over previous
import jax
import jax.numpy as jnp
from jax import lax
from jax.experimental import pallas as pl
from jax.experimental.pallas import tpu as pltpu
from jax.experimental.pallas import tpu_sc as plsc

BATCH = 16384
FIELDS = 26
EMBED = 64
TOTAL = BATCH * FIELDS
NUM_WORKERS = 32
PER_WORKER = TOTAL // NUM_WORKERS
CHUNK = 128
NCHUNKS = PER_WORKER // CHUNK
GROUP = 2
NGROUPS = NCHUNKS // GROUP
NBUF = 4
LOOKAHEAD = NBUF - 1


def _body(idx_hbm, table_hbm, out_hbm, idx_v, bufs, gsem, ssem):
    wid = lax.axis_index("s") * 2 + lax.axis_index("c")
    pltpu.sync_copy(idx_hbm.at[wid], idx_v)
    out_w = out_hbm.at[wid]

    def fire_gather(g, b):
        for i in range(GROUP):
            c = pl.multiple_of(g * GROUP + i, 1)
            pltpu.async_copy(table_hbm.at[idx_v.at[c]], bufs[b].at[i], gsem)

    def fire_store(g, b):
        gslc = pl.multiple_of(g * GROUP, GROUP)
        pltpu.async_copy(bufs[b], out_w.at[pl.ds(gslc, GROUP)], ssem)

    def wait_gather(b):
        for i in range(GROUP):
            pltpu.make_async_copy(table_hbm.at[idx_v.at[0]], bufs[b].at[i], gsem).wait()

    def wait_store(b):
        pltpu.make_async_copy(bufs[b], out_w.at[pl.ds(0, GROUP)], ssem).wait()

    for p in range(LOOKAHEAD):
        fire_gather(p, p)

    def outer(t, carry):
        g0 = t * NBUF
        for p in range(NBUF):
            g = g0 + p
            wait_gather(p)
            fire_store(g, p)
            @pl.when(g >= 1)
            def _():
                wait_store(p)

            @pl.when(g + LOOKAHEAD < NGROUPS)
            def _():
                fire_gather(g + LOOKAHEAD, (p + LOOKAHEAD) % NBUF)

        return carry

    lax.fori_loop(0, NGROUPS // NBUF, outer, 0)
    wait_store(0)


def kernel(indices, E):
    idx = indices.reshape(NUM_WORKERS, NCHUNKS, CHUNK).astype(jnp.int32)
    mesh = plsc.VectorSubcoreMesh(core_axis_name="c", subcore_axis_name="s")
    run = pl.kernel(
        _body,
        out_type=jax.ShapeDtypeStruct((NUM_WORKERS, NCHUNKS, CHUNK, EMBED), jnp.float32),
        mesh=mesh,
        scratch_types=[
            pltpu.VMEM((NCHUNKS, CHUNK), jnp.int32),
            [pltpu.VMEM((GROUP, CHUNK, EMBED), jnp.float32) for _ in range(NBUF)],
            pltpu.SemaphoreType.DMA,
            pltpu.SemaphoreType.DMA,
        ],
        compiler_params=pltpu.CompilerParams(use_tc_tiling_on_sc=False),
    )
    out = run(idx, E)
    return out.reshape(BATCH, FIELDS, EMBED)
